# baseline (device time: 140945 ns/iter reference)
import jax
import jax.numpy as jnp
from jax import lax
from jax.experimental import pallas as pl
from jax.experimental.pallas import tpu as pltpu

CS = 8
R = 4
CC = CS * R


def kernel(x):
    m, n = x.shape
    M = 2 * m
    half = m // 2
    chs = half // CS
    chc = half // CC

    def body(x_hbm, out_ref, xtmp, ltmp_sem, y_send, y_recv, x_send, x_recv):
        my_x = lax.axis_index("x")
        my_y = lax.axis_index("y")

        barrier = pltpu.get_barrier_semaphore()
        pl.semaphore_signal(barrier, inc=1, device_id=(my_x, 1 - my_y),
                            device_id_type=pl.DeviceIdType.MESH)
        pl.semaphore_signal(barrier, inc=1, device_id=(1 - my_x, my_y),
                            device_id_type=pl.DeviceIdType.MESH)
        pl.semaphore_wait(barrier, 2)

        def load(src_row, slot):
            return pltpu.make_async_copy(
                x_hbm.at[pl.ds(src_row, chs), :], xtmp.at[slot],
                ltmp_sem.at[slot])

        own0 = my_y * m + my_x * half
        fwd0 = (1 - my_y) * m + my_x * half

        def rdma_y(c):
            sl = pl.ds(own0 + c * chc, chc)
            return pltpu.make_async_remote_copy(
                src_ref=out_ref.at[sl, :], dst_ref=out_ref.at[sl, :],
                send_sem=y_send.at[c], recv_sem=y_recv.at[c],
                device_id=(my_x, 1 - my_y),
                device_id_type=pl.DeviceIdType.MESH)

        def rdma_x(c):
            sl = pl.ds(fwd0 + c * chc, chc)
            return pltpu.make_async_remote_copy(
                src_ref=out_ref.at[sl, :], dst_ref=out_ref.at[sl, :],
                send_sem=x_send.at[c], recv_sem=x_recv.at[c],
                device_id=(1 - my_x, my_y),
                device_id_type=pl.DeviceIdType.MESH)

        def stage_block(src_half_row, dst_global_row, b, last):
            load(src_half_row + b * chs, b % 2).wait()
            if not last:
                load(src_half_row + (b + 1) * chs, (b + 1) % 2).start()
            out_ref[pl.ds(dst_global_row + b * chs, chs), :] = (
                xtmp[b % 2].astype(out_ref.dtype))

        my_src = my_x * half
        load(my_src, 0).start()
        for b in range(CS):
            stage_block(my_src, own0, b, last=(b == CS - 1))
            for r in range(R):
                rdma_y(b * R + r).start()

        oth_src = (1 - my_x) * half
        oth0 = my_y * m + (1 - my_x) * half
        load(oth_src, 0).start()
        for c in range(CC):
            rdma_y(c).wait_recv()
            rdma_x(c).start()
            b = c // R
            if c % R == 0:
                load(oth_src + b * chs, b % 2).wait()
                if b + 1 < CS:
                    load(oth_src + (b + 1) * chs, (b + 1) % 2).start()
            out_ref[pl.ds(oth0 + c * chc, chc), :] = (
                xtmp[b % 2, pl.ds((c % R) * chc, chc), :].astype(
                    out_ref.dtype))

        for c in range(CC):
            rdma_x(c).wait_recv()
        for c in range(CC):
            rdma_y(c).wait_send()
            rdma_x(c).wait_send()

    return pl.pallas_call(
        body,
        out_shape=jax.ShapeDtypeStruct((M, n), jnp.bfloat16),
        in_specs=[pl.BlockSpec(memory_space=pl.ANY)],
        out_specs=pl.BlockSpec(memory_space=pltpu.VMEM),
        scratch_shapes=[
            pltpu.VMEM((2, half // CS, n), x.dtype),
            pltpu.SemaphoreType.DMA((2,)),
            pltpu.SemaphoreType.DMA((CC,)),
            pltpu.SemaphoreType.DMA((CC,)),
            pltpu.SemaphoreType.DMA((CC,)),
            pltpu.SemaphoreType.DMA((CC,)),
        ],
        compiler_params=pltpu.CompilerParams(
            collective_id=0, vmem_limit_bytes=60 * 1024 * 1024),
    )(x)
